# baseline (speedup 1.0000x reference)
import functools

import jax
import jax.numpy as jnp
from jax import lax
from jax.experimental import pallas as pl
from jax.experimental.pallas import tpu as pltpu
from jax.experimental.pallas import tpu_sc as plsc

WALKERS = 2

NC, NS, L = 2, 16, 16
CHUNK = 4
DEG = 32
D = 128
E = CHUNK * DEG
IQ = 128
NBUF = 4
NPW0 = 640
NPW1 = 0


def _scores_body(f_ref, a_ref, o_ref):
    o_ref[...] = jnp.dot(f_ref[...], a_ref[...],
                         preferred_element_type=jnp.float32)


def _compute_scores(node_features, att):
    n = node_features.shape[0]
    return pl.pallas_call(
        _scores_body,
        out_shape=jax.ShapeDtypeStruct((n, 2), jnp.float32),
    )(node_features, att)


def _sc_gnn(nbr2d, s_pad, r_pad, table, n_pad, n_nodes_tab):
    assert NPW0 % (CHUNK * NBUF) == 0 and NPW1 % (CHUNK * NBUF) == 0
    assert NS * (NPW0 + NPW1) == n_pad and E == IQ
    mesh = plsc.VectorSubcoreMesh(core_axis_name="c", subcore_axis_name="s")

    idx_rows = NPW0 * DEG // IQ

    @functools.partial(
        pl.kernel,
        mesh=mesh,
        out_type=jax.ShapeDtypeStruct((n_pad, D), jnp.float32),
        compiler_params=pltpu.CompilerParams(needs_layout_passes=False),
        scratch_types=[
            pltpu.VMEM((NPW0,), jnp.float32),
            pltpu.VMEM((n_pad,), jnp.float32),
            pltpu.VMEM((idx_rows, IQ), jnp.int32),
            [pltpu.VMEM((E, D), jnp.float32) for _ in range(NBUF)],
            pltpu.VMEM((E,), jnp.float32),
            pltpu.VMEM((NBUF * CHUNK, D), jnp.float32),
            [pltpu.SemaphoreType.DMA for _ in range(NBUF)],
        ],
    )
    def body(nbr_hbm, s_hbm, r_hbm, tab_hbm, out_hbm,
             s_v, r_v, idx_v, rows_bufs, attn_v, out_v, sems):
        cid = lax.axis_index("c")
        sid = lax.axis_index("s")
        n_chunks = NPW0 // CHUNK
        base_node = pl.multiple_of(sid * NPW0, 32)
        idx_row0 = pl.multiple_of(base_node * DEG // IQ, 8)
        toff = jnp.broadcast_to(cid * n_nodes_tab, (L,))

        def gather_cp(g, b):
            return pltpu.make_async_copy(
                tab_hbm.at[idx_v.at[g]], rows_bufs[b], sems[b])

        def compute(g, b, part):
            rows_ref = rows_bufs[b]
            ws = []
            for k in range(E // L):
                eidx = idx_v[g, pl.ds(k * L, L)] - toff
                rvec = plsc.load_gather(r_v, [eidx])
                nid = jnp.full((L,), k // 2, jnp.int32) + g * CHUNK
                svec = plsc.load_gather(s_v, [nid])
                y = svec + rvec
                t = 1.0 - 2.0 / (jnp.exp(2.0 * y) + 1.0)
                ws.append(jnp.exp(t))
            for i in range(CHUNK):
                denom = jnp.sum(ws[2 * i] + ws[2 * i + 1])
                scale = (jnp.full((L,), float(WALKERS), jnp.float32)
                         / jnp.broadcast_to(denom, (L,)))
                attn_v[pl.ds((2 * i) * L, L)] = ws[2 * i] * scale
                attn_v[pl.ds((2 * i + 1) * L, L)] = ws[2 * i + 1] * scale

            def ibody(i, carry):
                accs = [jnp.zeros((L,), jnp.float32) for _ in range(D // L)]
                for j in range(DEG):
                    e = i * DEG + j
                    avec = plsc.load_gather(
                        attn_v, [jnp.broadcast_to(e, (L,))])
                    for dv in range(D // L):
                        accs[dv] = accs[dv] + avec * rows_ref[
                            e, pl.ds(dv * L, L)]
                for dv in range(D // L):
                    out_v[part * CHUNK + i, pl.ds(dv * L, L)] = accs[dv]
                return carry
            lax.fori_loop(0, CHUNK, ibody, 0)

        def write_out(t):
            rows = NBUF * CHUNK
            pltpu.sync_copy(
                out_v, out_hbm.at[pl.ds(base_node + t * rows, rows), :])

        @pl.when(cid == 0)
        def _():
            cp_s = pltpu.make_async_copy(
                s_hbm.at[pl.ds(base_node, NPW0)], s_v, sems[0])
            cp_r = pltpu.make_async_copy(r_hbm, r_v, sems[1])
            cp_s.start()
            cp_r.start()
            pltpu.sync_copy(
                nbr_hbm.at[pl.ds(idx_row0, idx_rows), :], idx_v)
            cp_s.wait()
            cp_r.wait()

            for b in range(NBUF):
                gather_cp(b, b).start()

            def group_body(t, carry):
                g0 = NBUF * t
                for u in range(NBUF):
                    gather_cp(g0 + u, u).wait()
                    compute(g0 + u, u, u)
                    gather_cp(g0 + u + NBUF, u).start()
                write_out(t)
                return carry

            n_groups = n_chunks // NBUF
            lax.fori_loop(0, n_groups - 1, group_body, 0)
            gl = n_chunks - NBUF
            for u in range(NBUF):
                gather_cp(gl + u, u).wait()
                compute(gl + u, u, u)
            write_out(n_groups - 1)

    return body(nbr2d, s_pad, r_pad, table)


def kernel(node_features, neighbors, att_src, att_dst):
    n_nodes, deg = neighbors.shape
    assert deg == DEG and node_features.shape[1] == D
    n_pad = NS * (NPW0 + NPW1)
    assert n_pad >= n_nodes

    att = jnp.concatenate([att_src, att_dst], axis=1)
    scores = _compute_scores(node_features, att)
    pad = n_pad - n_nodes
    s_pad = jnp.pad(scores[:, 0], (0, pad + (NPW0 - NPW1)))
    r_pad = jnp.pad(scores[:, 1], (0, pad))
    nbr2d = jnp.pad(
        neighbors, ((0, pad + (NPW0 - NPW1)), (0, 0))).reshape(-1, IQ)

    tab2 = jnp.concatenate([node_features, node_features], axis=0)

    out = _sc_gnn(nbr2d, s_pad, r_pad, tab2, n_pad, n_nodes)
    return out[:n_nodes]


# speedup vs baseline: 1.0560x; 1.0560x over previous
import functools

import jax
import jax.numpy as jnp
from jax import lax
from jax.experimental import pallas as pl
from jax.experimental.pallas import tpu as pltpu
from jax.experimental.pallas import tpu_sc as plsc

WALKERS = 2

NC, NS, L = 2, 16, 16
CHUNK = 4
DEG = 32
D = 128
E = CHUNK * DEG
IQ = 128
NBUF = 4
NPW0 = 640
NPW1 = 0


def _scores_body(f_ref, a_ref, o_ref):
    o_ref[...] = jnp.dot(f_ref[...], a_ref[...],
                         preferred_element_type=jnp.float32)


def _compute_scores(node_features, att):
    n = node_features.shape[0]
    return pl.pallas_call(
        _scores_body,
        out_shape=jax.ShapeDtypeStruct((n, 2), jnp.float32),
    )(node_features, att)


def _sc_gnn(nbr2d, s_pad, r_pad, table, n_pad, n_nodes_tab):
    assert NPW0 % (CHUNK * NBUF) == 0 and NPW1 % (CHUNK * NBUF) == 0
    assert NS * (NPW0 + NPW1) == n_pad and E == IQ
    mesh = plsc.VectorSubcoreMesh(core_axis_name="c", subcore_axis_name="s")

    idx_rows = NPW0 * DEG // IQ // 2

    @functools.partial(
        pl.kernel,
        mesh=mesh,
        out_type=jax.ShapeDtypeStruct((n_pad, D), jnp.float32),
        compiler_params=pltpu.CompilerParams(needs_layout_passes=False),
        scratch_types=[
            pltpu.VMEM((NPW0,), jnp.float32),
            pltpu.VMEM((n_pad,), jnp.float32),
            pltpu.VMEM((idx_rows, IQ), jnp.int32),
            [pltpu.VMEM((E, D), jnp.float32) for _ in range(NBUF)],
            pltpu.VMEM((E,), jnp.float32),
            pltpu.VMEM((NBUF * CHUNK, D), jnp.float32),
            [pltpu.SemaphoreType.DMA for _ in range(NBUF)],
        ],
    )
    def body(nbr_hbm, s_hbm, r_hbm, tab_hbm, out_hbm,
             s_v, r_v, idx_v, rows_bufs, attn_v, out_v, sems):
        cid = lax.axis_index("c")
        sid = lax.axis_index("s")
        n_chunks = NPW0 // CHUNK
        base_node = pl.multiple_of(sid * NPW0, 32)
        idx_row0 = pl.multiple_of(base_node * DEG // IQ, 8)
        toff = jnp.broadcast_to(cid * n_nodes_tab, (L,))

        def gather_cp(g, b):
            return pltpu.make_async_copy(
                tab_hbm.at[idx_v.at[g]], rows_bufs[b], sems[b])

        def compute(g, b, part, noff):
            rows_ref = rows_bufs[b]
            ws = []
            for k in range(E // L):
                eidx = idx_v[g, pl.ds(k * L, L)] - toff
                rvec = plsc.load_gather(r_v, [eidx])
                nid = jnp.full((L,), k // 2, jnp.int32) + (g * CHUNK + noff)
                svec = plsc.load_gather(s_v, [nid])
                y = svec + rvec
                t = 1.0 - 2.0 / (jnp.exp(2.0 * y) + 1.0)
                ws.append(jnp.exp(t))
            for i in range(CHUNK):
                denom = jnp.sum(ws[2 * i] + ws[2 * i + 1])
                scale = (jnp.full((L,), float(WALKERS), jnp.float32)
                         / jnp.broadcast_to(denom, (L,)))
                attn_v[pl.ds((2 * i) * L, L)] = ws[2 * i] * scale
                attn_v[pl.ds((2 * i + 1) * L, L)] = ws[2 * i + 1] * scale

            JU = 4

            def ibody(i, carry):
                def jbody(jj, accs):
                    accs = list(accs)
                    for v in range(JU):
                        e = i * DEG + jj * JU + v
                        avec = plsc.load_gather(
                            attn_v, [jnp.broadcast_to(e, (L,))])
                        for dv in range(D // L):
                            accs[dv] = accs[dv] + avec * rows_ref[
                                e, pl.ds(dv * L, L)]
                    return tuple(accs)
                accs = lax.fori_loop(
                    0, DEG // JU, jbody,
                    tuple(jnp.zeros((L,), jnp.float32)
                          for _ in range(D // L)))
                for dv in range(D // L):
                    out_v[part * CHUNK + i, pl.ds(dv * L, L)] = accs[dv]
                return carry
            lax.fori_loop(0, CHUNK, ibody, 0)

        @pl.when(cid == 0)
        def _():
            cp_s = pltpu.make_async_copy(
                s_hbm.at[pl.ds(base_node, NPW0)], s_v, sems[0])
            cp_r = pltpu.make_async_copy(r_hbm, r_v, sems[1])
            cp_s.start()
            cp_r.start()
            nh = NPW0 // 2
            ch = nh // CHUNK
            for half in range(2):
                hbase = pl.multiple_of(base_node + half * nh, 32)
                pltpu.sync_copy(
                    nbr_hbm.at[pl.ds(idx_row0 + half * idx_rows,
                                     idx_rows), :], idx_v)
                if half == 0:
                    cp_s.wait()
                    cp_r.wait()
                noff = half * nh

                def write_out(t):
                    rows = NBUF * CHUNK
                    pltpu.sync_copy(
                        out_v, out_hbm.at[pl.ds(hbase + t * rows, rows), :])

                for b in range(NBUF):
                    gather_cp(b, b).start()

                def group_body(t, carry):
                    g0 = NBUF * t
                    for u in range(NBUF):
                        gather_cp(g0 + u, u).wait()
                        compute(g0 + u, u, u, noff)
                        gather_cp(g0 + u + NBUF, u).start()
                    write_out(t)
                    return carry

                n_groups = ch // NBUF
                lax.fori_loop(0, n_groups - 1, group_body, 0)
                gl = ch - NBUF
                for u in range(NBUF):
                    gather_cp(gl + u, u).wait()
                    compute(gl + u, u, u, noff)
                write_out(n_groups - 1)

    return body(nbr2d, s_pad, r_pad, table)


def kernel(node_features, neighbors, att_src, att_dst):
    n_nodes, deg = neighbors.shape
    assert deg == DEG and node_features.shape[1] == D
    n_pad = NS * (NPW0 + NPW1)
    assert n_pad >= n_nodes

    att = jnp.concatenate([att_src, att_dst], axis=1)
    scores = _compute_scores(node_features, att)
    pad = n_pad - n_nodes
    s_pad = jnp.pad(scores[:, 0], (0, pad + (NPW0 - NPW1)))
    r_pad = jnp.pad(scores[:, 1], (0, pad))
    nbr2d = jnp.pad(
        neighbors, ((0, pad + (NPW0 - NPW1)), (0, 0))).reshape(-1, IQ)

    tab2 = jnp.concatenate([node_features, node_features], axis=0)

    out = _sc_gnn(nbr2d, s_pad, r_pad, tab2, n_pad, n_nodes)
    return out[:n_nodes]


# speedup vs baseline: 1.7153x; 1.6244x over previous
import functools

import jax
import jax.numpy as jnp
from jax import lax
from jax.experimental import pallas as pl
from jax.experimental.pallas import tpu as pltpu
from jax.experimental.pallas import tpu_sc as plsc

WALKERS = 2

NC, NS, L = 2, 16, 16
CHUNK = 4
DEG = 32
D = 128
E = CHUNK * DEG
IQ = 128
NBUF = 4
NPW0 = 608
NPW1 = 32


def _scores_body(f_ref, a_ref, o_ref):
    o_ref[...] = jnp.dot(f_ref[...], a_ref[...],
                         preferred_element_type=jnp.float32)


def _compute_scores(node_features, att):
    n = node_features.shape[0]
    return pl.pallas_call(
        _scores_body,
        out_shape=jax.ShapeDtypeStruct((n, 2), jnp.float32),
    )(node_features, att)


def _sc_gnn(nbr2d, s_pad, r_pad, table, n_pad, n_nodes_tab):
    assert NPW0 % (CHUNK * NBUF) == 0 and NPW1 % (CHUNK * NBUF) == 0
    assert NS * (NPW0 + NPW1) == n_pad and E == IQ
    mesh = plsc.VectorSubcoreMesh(core_axis_name="c", subcore_axis_name="s")

    idx_rows = NPW0 * DEG // IQ

    @functools.partial(
        pl.kernel,
        mesh=mesh,
        out_type=jax.ShapeDtypeStruct((n_pad, D), jnp.float32),
        compiler_params=pltpu.CompilerParams(needs_layout_passes=False),
        scratch_types=[
            pltpu.VMEM((NPW0,), jnp.float32),
            pltpu.VMEM((n_pad,), jnp.float32),
            pltpu.VMEM((idx_rows, IQ), jnp.int32),
            [pltpu.VMEM((E, D), jnp.float32) for _ in range(NBUF)],
            pltpu.VMEM((E,), jnp.float32),
            pltpu.VMEM((NBUF * CHUNK, D), jnp.float32),
            [pltpu.SemaphoreType.DMA for _ in range(NBUF)],
        ],
    )
    def body(nbr_hbm, s_hbm, r_hbm, tab_hbm, out_hbm,
             s_v, r_v, idx_v, rows_bufs, attn_v, out_v, sems):
        cid = lax.axis_index("c")
        sid = lax.axis_index("s")
        base_node = pl.multiple_of(
            jnp.where(cid == 0, sid * NPW0, NS * NPW0 + sid * NPW1), 32)
        n_chunks = jnp.where(cid == 0, NPW0 // CHUNK, NPW1 // CHUNK)
        idx_row0 = pl.multiple_of(base_node * DEG // IQ, 8)
        cp_s = pltpu.make_async_copy(
            s_hbm.at[pl.ds(base_node, NPW0)], s_v, sems[0])
        cp_r = pltpu.make_async_copy(r_hbm, r_v, sems[1])
        cp_s.start()
        cp_r.start()
        idx_rows1 = NPW1 * DEG // IQ

        @pl.when(cid == 0)
        def _():
            pltpu.sync_copy(
                nbr_hbm.at[pl.ds(idx_row0, idx_rows), :], idx_v)

        @pl.when(cid != 0)
        def _():
            pltpu.sync_copy(
                nbr_hbm.at[pl.ds(idx_row0, idx_rows1), :],
                idx_v.at[pl.ds(0, idx_rows1), :])
        cp_s.wait()
        cp_r.wait()
        toff = jnp.broadcast_to(cid * n_nodes_tab, (L,))

        def adj_body(rr, carry):
            for o in range(IQ // L):
                sl = pl.ds(o * L, L)
                idx_v[rr, sl] = idx_v[rr, sl] + toff
            return carry
        lax.fori_loop(0, idx_rows, adj_body, 0)

        def gather_cp(g, b):
            return pltpu.make_async_copy(
                tab_hbm.at[idx_v.at[g]], rows_bufs[b], sems[b])

        def compute(g, b, part):
            rows_ref = rows_bufs[b]
            ws = []
            for k in range(E // L):
                eidx = idx_v[g, pl.ds(k * L, L)] - toff
                rvec = plsc.load_gather(r_v, [eidx])
                nid = jnp.full((L,), k // 2, jnp.int32) + g * CHUNK
                svec = plsc.load_gather(s_v, [nid])
                y = svec + rvec
                t = 1.0 - 2.0 / (jnp.exp(2.0 * y) + 1.0)
                ws.append(jnp.exp(t))
            for i in range(CHUNK):
                denom = jnp.sum(ws[2 * i] + ws[2 * i + 1])
                scale = (jnp.full((L,), float(WALKERS), jnp.float32)
                         / jnp.broadcast_to(denom, (L,)))
                attn_v[pl.ds((2 * i) * L, L)] = ws[2 * i] * scale
                attn_v[pl.ds((2 * i + 1) * L, L)] = ws[2 * i + 1] * scale

            def ibody(i, carry):
                accs = [jnp.zeros((L,), jnp.float32) for _ in range(D // L)]
                for j in range(DEG):
                    e = i * DEG + j
                    avec = plsc.load_gather(
                        attn_v, [jnp.broadcast_to(e, (L,))])
                    for dv in range(D // L):
                        accs[dv] = accs[dv] + avec * rows_ref[
                            e, pl.ds(dv * L, L)]
                for dv in range(D // L):
                    out_v[part * CHUNK + i, pl.ds(dv * L, L)] = accs[dv]
                return carry
            lax.fori_loop(0, CHUNK, ibody, 0)

        def write_out(t):
            rows = NBUF * CHUNK
            pltpu.sync_copy(
                out_v, out_hbm.at[pl.ds(base_node + t * rows, rows), :])

        for b in range(NBUF):
            gather_cp(b, b).start()

        def group_body(t, carry):
            g0 = NBUF * t
            for u in range(NBUF):
                gather_cp(g0 + u, u).wait()
                compute(g0 + u, u, u)
                gather_cp(g0 + u + NBUF, u).start()
            write_out(t)
            return carry

        n_groups = n_chunks // NBUF
        lax.fori_loop(0, n_groups - 1, group_body, 0)
        gl = n_chunks - NBUF
        for u in range(NBUF):
            gather_cp(gl + u, u).wait()
            compute(gl + u, u, u)
        write_out(n_groups - 1)

    return body(nbr2d, s_pad, r_pad, table)


def kernel(node_features, neighbors, att_src, att_dst):
    n_nodes, deg = neighbors.shape
    assert deg == DEG and node_features.shape[1] == D
    n_pad = NS * (NPW0 + NPW1)
    assert n_pad >= n_nodes

    att = jnp.concatenate([att_src, att_dst], axis=1)
    scores = _compute_scores(node_features, att)
    pad = n_pad - n_nodes
    s_pad = jnp.pad(scores[:, 0], (0, pad + (NPW0 - NPW1)))
    r_pad = jnp.pad(scores[:, 1], (0, pad))
    nbr2d = jnp.pad(
        neighbors, ((0, pad + (NPW0 - NPW1)), (0, 0))).reshape(-1, IQ)

    tab2 = jnp.concatenate([node_features, node_features], axis=0)

    out = _sc_gnn(nbr2d, s_pad, r_pad, tab2, n_pad, n_nodes)
    return out[:n_nodes]
